# baseline (speedup 1.0000x reference)
import functools

import jax
import jax.numpy as jnp
from jax import lax
from jax.experimental import pallas as pl
from jax.experimental.pallas import tpu as pltpu
from jax.experimental.pallas import tpu_sc as plsc

VOCAB = 64001
DIM = 128
HALF = DIM // 2
MAX_LEN = 200
B = 4096
TOK = B * MAX_LEN
EPS = 1e-12


def _transform_body(w_ref, w2_ref, o_ref):
    t = lax.dot_general(
        w_ref[...], w2_ref[...], (((1,), (1,)), ((), ())),
        preferred_element_type=jnp.float32)
    tb = t.astype(jnp.bfloat16)
    a = lax.bitcast_convert_type(tb[:, :HALF], jnp.uint16).astype(jnp.uint32)
    b = lax.bitcast_convert_type(tb[:, HALF:], jnp.uint16).astype(jnp.uint32)
    o_ref[...] = lax.bitcast_convert_type((b << 16) | a, jnp.int32)


def _transform_table(word_table, W2):
    R = 512
    return pl.pallas_call(
        _transform_body,
        grid=(pl.cdiv(VOCAB, R),),
        in_specs=[pl.BlockSpec((R, DIM), lambda i: (i, 0)),
                  pl.BlockSpec((DIM, DIM), lambda i: (0, 0))],
        out_specs=pl.BlockSpec((R, HALF), lambda i: (i, 0)),
        out_shape=jax.ShapeDtypeStruct((VOCAB, HALF), jnp.int32),
    )(word_table, W2)


_NW = 32
_B_PER_W = TOK // _NW
_CH = 256
_IR = _CH // 128
_NCH = _B_PER_W // _CH


def _sc_gather(table, ids2d):
    mesh = plsc.VectorSubcoreMesh(core_axis_name="c", subcore_axis_name="s")

    @functools.partial(
        pl.kernel,
        out_type=jax.ShapeDtypeStruct((TOK, HALF), jnp.int32),
        mesh=mesh,
        scratch_types=[
            pltpu.VMEM((2, _IR, 128), jnp.int32),
            pltpu.VMEM((2, _CH, HALF), jnp.int32),
            pltpu.SemaphoreType.DMA,
            pltpu.SemaphoreType.DMA,
            pltpu.SemaphoreType.DMA,
            pltpu.SemaphoreType.DMA,
        ],
        compiler_params=pltpu.CompilerParams(use_tc_tiling_on_sc=False),
    )
    def k(t_hbm, ids_hbm, out_hbm, idx_v, rows_v, sg0, sg1, so0, so1):
        wid = lax.axis_index("s") * 2 + lax.axis_index("c")
        sg = (sg0, sg1)
        so = (so0, so1)

        def issue(g, b):
            irow = wid * (_B_PER_W // 128) + g * _IR
            pltpu.sync_copy(ids_hbm.at[pl.ds(irow, _IR)], idx_v.at[b])
            for j in range(_IR):
                pltpu.async_copy(t_hbm.at[idx_v.at[b].at[j]],
                                 rows_v.at[b].at[pl.ds(j * 128, 128)], sg[b])

        def flush(g, b):
            for j in range(_IR):
                pltpu.make_async_copy(t_hbm.at[idx_v.at[b].at[j]],
                                      rows_v.at[b].at[pl.ds(j * 128, 128)],
                                      sg[b]).wait()
            base = wid * _B_PER_W + g * _CH
            pltpu.async_copy(rows_v.at[b], out_hbm.at[pl.ds(base, _CH)], so[b])

        def wait_out(b):
            pltpu.make_async_copy(rows_v.at[b], out_hbm.at[pl.ds(0, _CH)],
                                  so[b]).wait()

        issue(0, 0)

        def body(i, carry):
            g0 = 2 * i

            @pl.when(i >= 1)
            def _():
                wait_out(1)
            issue(g0 + 1, 1)
            flush(g0, 0)
            wait_out(0)

            @pl.when(g0 + 2 < _NCH)
            def _():
                issue(g0 + 2, 0)
            flush(g0 + 1, 1)
            return carry

        lax.fori_loop(0, _NCH // 2, body, 0)
        wait_out(1)

    return k(table, ids2d)


_BR = 16
_N = _BR * MAX_LEN


def _ln_body(g_ref, seg_ref, poslo_ref, poshi_ref, typlo_ref, typhi_ref,
             gamlo_ref, gamhi_ref, betlo_ref, bethi_ref, o_ref):
    g = g_ref[...]
    lo = lax.bitcast_convert_type(g << 16, jnp.float32)
    hi = lax.bitcast_convert_type(g & jnp.int32(-65536), jnp.float32)
    seg = seg_ref[...].reshape(_N, 1)
    oneh = (seg == lax.broadcasted_iota(jnp.int32, (_N, 8), 1)
            ).astype(jnp.float32)
    tlo = lax.dot_general(oneh, typlo_ref[...], (((1,), (0,)), ((), ())),
                          preferred_element_type=jnp.float32)
    thi = lax.dot_general(oneh, typhi_ref[...], (((1,), (0,)), ((), ())),
                          preferred_element_type=jnp.float32)
    xlo = (lo + poslo_ref[...][None, :, :]).reshape(_N, HALF) + tlo
    xhi = (hi + poshi_ref[...][None, :, :]).reshape(_N, HALF) + thi
    ones = jnp.ones((HALF, 1), jnp.float32)
    dot = lambda a: lax.dot_general(a, ones, (((1,), (0,)), ((), ())),
                                    preferred_element_type=jnp.float32)
    ssum = dot(xlo) + dot(xhi)
    ssq = dot(xlo * xlo) + dot(xhi * xhi)
    mean = ssum * (1.0 / DIM)
    var = ssq * (1.0 / DIM) - mean * mean
    r = lax.rsqrt(var + EPS)
    ylo = (xlo - mean) * r * gamlo_ref[...] + betlo_ref[...]
    yhi = (xhi - mean) * r * gamhi_ref[...] + bethi_ref[...]
    y = jnp.concatenate([ylo.reshape(_BR, MAX_LEN, HALF),
                         yhi.reshape(_BR, MAX_LEN, HALF)], axis=-1)
    o_ref[...] = y


def _ln(gathered, segment_ids, pos_table, type_table, gamma, beta):
    full = lambda shape: pl.BlockSpec(shape, lambda i: tuple(0 for _ in shape))
    typ8 = jnp.zeros((8, DIM), jnp.float32).at[:3].set(type_table)
    return pl.pallas_call(
        _ln_body,
        grid=(B // _BR,),
        in_specs=[
            pl.BlockSpec((_BR, MAX_LEN, HALF), lambda i: (i, 0, 0)),
            pl.BlockSpec((_BR, MAX_LEN, 1), lambda i: (i, 0, 0)),
            full((MAX_LEN, HALF)), full((MAX_LEN, HALF)),
            full((8, HALF)), full((8, HALF)),
            full((1, HALF)), full((1, HALF)),
            full((1, HALF)), full((1, HALF)),
        ],
        out_specs=pl.BlockSpec((_BR, MAX_LEN, DIM), lambda i: (i, 0, 0)),
        out_shape=jax.ShapeDtypeStruct((B, MAX_LEN, DIM), jnp.float32),
    )(gathered, segment_ids.reshape(B, MAX_LEN, 1),
      pos_table[:, :HALF], pos_table[:, HALF:],
      typ8[:, :HALF], typ8[:, HALF:],
      gamma[:HALF].reshape(1, HALF), gamma[HALF:].reshape(1, HALF),
      beta[:HALF].reshape(1, HALF), beta[HALF:].reshape(1, HALF))


def kernel(input_ids, segment_ids, word_table, W2, pos_table, type_table,
           gamma, beta):
    table = _transform_table(word_table, W2)
    ids2d = input_ids.astype(jnp.int32).reshape(TOK // 128, 128)
    gathered = _sc_gather(table, ids2d)
    return _ln(gathered.reshape(B, MAX_LEN, HALF), segment_ids.astype(jnp.int32),
               pos_table, type_table, gamma, beta)
